# baseline (speedup 1.0000x reference)
import numpy as np
import jax
import jax.numpy as jnp
from jax.experimental import pallas as pl
from jax.experimental.pallas import tpu as pltpu

_V = 46
_VP = 64
_CFG = [(2, 64, 1, False), (64, 64, 1, True), (64, 64, 1, True), (64, 64, 1, True),
        (64, 128, 2, True), (128, 128, 1, True), (128, 128, 1, True),
        (128, 256, 2, True), (256, 256, 1, True), (256, 256, 1, True)]


def _pool_interp_matrix(tq, tout, v, vp):
    s = np.zeros((tq * vp, tq), np.float64)
    for t in range(tq):
        s[t * vp:t * vp + v, t] = 1.0 / v
    pos = (np.arange(tout, dtype=np.float32) * np.float32(tq - 1)
           / np.float32(tout - 1)).astype(np.float64)
    lo = np.floor(pos).astype(np.int64)
    hi = np.clip(lo + 1, 0, tq - 1)
    w = pos - lo
    m = np.zeros((tq, tout), np.float64)
    m[lo, np.arange(tout)] += 1.0 - w
    m[hi, np.arange(tout)] += w
    return (s @ m).astype(np.float32)


_P = _pool_interp_matrix(16, 64, _V, _VP)


def _dot(a, b):
    return jnp.dot(a, b, preferred_element_type=jnp.float32)


def _sub2(a, t):
    c = a.shape[1]
    return a.reshape(t // 2, 2 * _VP, c)[:, :_VP, :].reshape((t // 2) * _VP, c)


def _unit(x, nxt, ci, co, it, t, st, has_down, res_kind):
        tv = t * _VP
        pa_r, wab_r, wdc_r, bd_r, gg_r, gb_r = (nxt() for _ in range(6))
        if has_down:
            dw_r, db_r, dg_r, dbb_r = (nxt() for _ in range(4))
        wt_r, tb_r, tg_r, tbe_r = (nxt() for _ in range(4))
        if res_kind == 'conv':
            rw_r, rb_r, rg_r, rbe_r = (nxt() for _ in range(4))

        aall = _dot(x, wab_r[...])
        a3 = jnp.transpose(aall.reshape(t, _VP, 6 * it), (0, 2, 1))
        xf = jnp.transpose(x.reshape(t, _VP, ci), (0, 2, 1)).reshape(t * ci, _VP)

        ads = []
        for i in range(3):
            a1 = a3[:, i * it:(i + 1) * it, :].reshape(t * it, _VP)
            a2 = a3[:, (3 + i) * it:(4 + i) * it, :].reshape(t * it, _VP)
            m = jax.lax.dot_general(a1, a2, (((0,), (0,)), ((), ())),
                                    preferred_element_type=jnp.float32)
            row = jax.lax.broadcasted_iota(jnp.int32, (_VP, _VP), 0)
            col = jax.lax.broadcasted_iota(jnp.int32, (_VP, _VP), 1)
            m = jnp.where(row < _V, m * (1.0 / (it * t)), -1e30)
            ads.append(jnp.where(col < _V, jax.nn.softmax(m, axis=0) + pa_r[i], 0.0))
        adc = jnp.concatenate(ads, axis=1)

        xa = _dot(xf, adc)
        xa3 = jnp.transpose(xa.reshape(t, ci, 3 * _VP), (0, 2, 1))
        xac = jnp.concatenate(
            [xa3[:, i * _VP:(i + 1) * _VP, :].reshape(tv, ci) for i in range(3)],
            axis=1)
        lin = _dot(xac, wdc_r[...])
        bsum = bd_r[0] + bd_r[1] + bd_r[2]
        out = (lin + bsum) * gg_r[...] + gb_r[...]
        if has_down:
            res = (_dot(x, dw_r[...]) + db_r[...]) * dg_r[...] + dbb_r[...]
        else:
            res = x
        y = jnp.maximum(out + res, 0.0)

        zpad = jnp.zeros((4 * _VP, co), jnp.float32)
        z = jnp.concatenate([zpad, y, zpad], axis=0)
        if st == 1:
            zc = jnp.concatenate([z[k * _VP:k * _VP + tv, :] for k in range(9)],
                                 axis=1)
        else:
            tow = (t // 2) * _VP
            z3 = z.reshape((t + 8) // 2, 2 * _VP, co)
            ze = z3[:, :_VP, :].reshape(((t + 8) // 2) * _VP, co)
            zo = z3[:, _VP:, :].reshape(((t + 8) // 2) * _VP, co)
            zc = jnp.concatenate(
                [ze[j * _VP:j * _VP + tow, :] for j in range(5)]
                + [zo[j * _VP:j * _VP + tow, :] for j in range(4)],
                axis=1)
        ytc = (_dot(zc, wt_r[...]) + tb_r[...]) * tg_r[...] + tbe_r[...]
        if res_kind == 'none':
            res2 = 0.0
        elif res_kind == 'id':
            res2 = x
        else:
            xs = _sub2(x, t) if st == 2 else x
            res2 = (_dot(xs, rw_r[...]) + rb_r[...]) * rg_r[...] + rbe_r[...]
        return jnp.maximum(ytc + res2, 0.0)


def _mega_body(*refs):
    pos = [0]

    def nxt():
        r = refs[pos[0]]
        pos[0] += 1
        return r

    x_ref = nxt()
    g0_r, b0_r = nxt(), nxt()
    x = x_ref[0] * g0_r[...] + b0_r[...]
    t = 64
    for ci, co, st, res in _CFG:
        has_down = ci != co
        res_kind = 'none' if not res else ('conv' if (ci != co or st != 1) else 'id')
        x = _unit(x, nxt, ci, co, co // 4, t, st, has_down, res_kind)
        t //= st
    p_r = nxt()
    refs[-1][0] = jax.lax.dot_general(p_r[...], x, (((0,), (0,)), ((), ())),
                                      preferred_element_type=jnp.float32)


def _full_spec(shape):
    nd = len(shape)
    return pl.BlockSpec(shape, lambda n: (0,) * nd)


def _layer_ins(layer, cfg):
    ci, co, st, res = cfg
    it = co // 4
    gcn, tcn = layer['gcn'], layer['tcn']
    res_kind = 'none' if not res else ('conv' if 'res' in layer else 'id')

    pa = jnp.zeros((3, _VP, _VP), jnp.float32).at[:, :_V, :_V].set(gcn['PA'])
    wab = jnp.concatenate([gcn['wa'].reshape(3 * it, ci),
                           gcn['wb'].reshape(3 * it, ci)], axis=0).T
    wdc = jnp.transpose(gcn['wd'].reshape(3, co, ci), (0, 2, 1)).reshape(3 * ci, co)
    ins = [pa, wab, wdc,
           gcn['bd'].reshape(3, 1, co),
           gcn['bn_g'].reshape(1, co),
           gcn['bn_b'].reshape(1, co)]
    if ci != co:
        ins += [gcn['down_w'].reshape(co, ci).T, gcn['down_b'].reshape(1, co),
                gcn['down_g'].reshape(1, co), gcn['down_bb'].reshape(1, co)]
    wt9 = jnp.transpose(tcn['w'].reshape(co, co, 9), (2, 1, 0))
    if st == 2:
        wt9 = wt9[jnp.asarray([0, 2, 4, 6, 8, 1, 3, 5, 7])]
    ins += [wt9.reshape(9 * co, co), tcn['b'].reshape(1, co),
            tcn['g'].reshape(1, co), tcn['be'].reshape(1, co)]
    if res_kind == 'conv':
        r = layer['res']
        ins += [r['w'].reshape(co, ci).T, r['b'].reshape(1, co),
                r['g'].reshape(1, co), r['be'].reshape(1, co)]
    return ins


def kernel(keypoints, params):
    n = keypoints.shape[0]
    xr = keypoints[..., 0]
    xp = jnp.pad(xr, ((0, 0), (0, 0), (0, 0), (0, _VP - _V)))
    x = jnp.transpose(xp, (0, 2, 3, 1)).reshape(n, 64 * _VP, 2)
    g2 = params['data_bn_g'].reshape(_V, 2)
    b2 = params['data_bn_b'].reshape(_V, 2)
    gv = jnp.pad(g2, ((0, _VP - _V), (0, 0)))
    bv = jnp.pad(b2, ((0, _VP - _V), (0, 0)))
    g0 = jnp.tile(gv, (64, 1))
    b0 = jnp.tile(bv, (64, 1))

    ins = [x, g0, b0]
    for layer, cfg in zip(params['layers'], _CFG):
        ins += _layer_ins(layer, cfg)
    ins += [jnp.asarray(_P)]

    in_specs = [pl.BlockSpec((1, 64 * _VP, 2), lambda n: (n, 0, 0))]
    in_specs += [_full_spec(a.shape) for a in ins[1:]]
    out_shape = jax.ShapeDtypeStruct((n, 64, 256), jnp.float32)
    out_specs = pl.BlockSpec((1, 64, 256), lambda n: (n, 0, 0))

    return pl.pallas_call(
        _mega_body,
        grid=(n,),
        in_specs=in_specs,
        out_specs=out_specs,
        out_shape=out_shape,
        compiler_params=pltpu.CompilerParams(
            dimension_semantics=("arbitrary",)),
    )(*ins)


# speedup vs baseline: 1.1317x; 1.1317x over previous
import numpy as np
import jax
import jax.numpy as jnp
from jax.experimental import pallas as pl
from jax.experimental.pallas import tpu as pltpu

_V = 46
_VP = 64
_CFG = [(2, 64, 1, False), (64, 64, 1, True), (64, 64, 1, True), (64, 64, 1, True),
        (64, 128, 2, True), (128, 128, 1, True), (128, 128, 1, True),
        (128, 256, 2, True), (256, 256, 1, True), (256, 256, 1, True)]


def _pool_interp_matrix(tq, tout, v, vp):
    s = np.zeros((tq * vp, tq), np.float64)
    for t in range(tq):
        s[t * vp:t * vp + v, t] = 1.0 / v
    pos = (np.arange(tout, dtype=np.float32) * np.float32(tq - 1)
           / np.float32(tout - 1)).astype(np.float64)
    lo = np.floor(pos).astype(np.int64)
    hi = np.clip(lo + 1, 0, tq - 1)
    w = pos - lo
    m = np.zeros((tq, tout), np.float64)
    m[lo, np.arange(tout)] += 1.0 - w
    m[hi, np.arange(tout)] += w
    return (s @ m).astype(np.float32)


_P = _pool_interp_matrix(16, 64, _V, _VP)


def _dot(a, b):
    return jnp.dot(a.astype(jnp.bfloat16), b.astype(jnp.bfloat16),
                   preferred_element_type=jnp.float32)


def _sub2(a, t):
    c = a.shape[1]
    return a.reshape(t // 2, 2 * _VP, c)[:, :_VP, :].reshape((t // 2) * _VP, c)


def _make_body(ci, co, it, t, st, has_down, res_kind, first, last):
    tv = t * _VP

    def body(*refs):
        pos = [0]

        def nxt():
            r = refs[pos[0]]
            pos[0] += 1
            return r

        x_ref = nxt()
        pa_r, wab_r, wdc_r, bd_r, gg_r, gb_r = (nxt() for _ in range(6))
        if has_down:
            dw_r, db_r, dg_r, dbb_r = (nxt() for _ in range(4))
        wt_r, tb_r, tg_r, tbe_r = (nxt() for _ in range(4))
        if res_kind == 'conv':
            rw_r, rb_r, rg_r, rbe_r = (nxt() for _ in range(4))
        if first:
            g0_r, b0_r = nxt(), nxt()
        if last:
            p_r = nxt()
        o_ref = refs[-1]

        x = x_ref[0]
        if first:
            x = x * g0_r[...] + b0_r[...]

        aall = _dot(x, wab_r[...])
        a3 = jnp.transpose(aall.reshape(t, _VP, 6 * it), (0, 2, 1))
        xf = jnp.transpose(x.reshape(t, _VP, ci), (0, 2, 1)).reshape(t * ci, _VP)

        ms = []
        for i in range(3):
            a1 = a3[:, i * it:(i + 1) * it, :].reshape(t * it, _VP)
            a2 = a3[:, (3 + i) * it:(4 + i) * it, :].reshape(t * it, _VP)
            ms.append(jax.lax.dot_general(a1, a2, (((0,), (0,)), ((), ())),
                                          preferred_element_type=jnp.float32))
        mc = jnp.concatenate(ms, axis=1)
        row = jax.lax.broadcasted_iota(jnp.int32, (_VP, 3 * _VP), 0)
        col = jax.lax.broadcasted_iota(jnp.int32, (_VP, 3 * _VP), 1)
        mc = jnp.where(row < _V, mc * (1.0 / (it * t)), -1e30)
        adc = jnp.where(col % _VP < _V,
                        jax.nn.softmax(mc, axis=0) + pa_r[...], 0.0)

        xa = _dot(xf, adc)
        xa3 = jnp.transpose(xa.reshape(t, ci, 3 * _VP), (0, 2, 1))
        xac = jnp.concatenate(
            [xa3[:, i * _VP:(i + 1) * _VP, :].reshape(tv, ci) for i in range(3)],
            axis=1)
        lin = _dot(xac, wdc_r[...])
        bsum = bd_r[0] + bd_r[1] + bd_r[2]
        out = (lin + bsum) * gg_r[...] + gb_r[...]
        if has_down:
            res = (_dot(x, dw_r[...]) + db_r[...]) * dg_r[...] + dbb_r[...]
        else:
            res = x
        y = jnp.maximum(out + res, 0.0)

        zpad = jnp.zeros((4 * _VP, co), jnp.float32)
        z = jnp.concatenate([zpad, y, zpad], axis=0)
        if st == 1:
            zc = jnp.concatenate([z[k * _VP:k * _VP + tv, :] for k in range(9)],
                                 axis=1)
        else:
            tow = (t // 2) * _VP
            z3 = z.reshape((t + 8) // 2, 2 * _VP, co)
            ze = z3[:, :_VP, :].reshape(((t + 8) // 2) * _VP, co)
            zo = z3[:, _VP:, :].reshape(((t + 8) // 2) * _VP, co)
            zc = jnp.concatenate(
                [ze[j * _VP:j * _VP + tow, :] for j in range(5)]
                + [zo[j * _VP:j * _VP + tow, :] for j in range(4)],
                axis=1)
        ytc = (_dot(zc, wt_r[...]) + tb_r[...]) * tg_r[...] + tbe_r[...]
        if res_kind == 'none':
            res2 = 0.0
        elif res_kind == 'id':
            res2 = x
        else:
            xs = _sub2(x, t) if st == 2 else x
            res2 = (_dot(xs, rw_r[...]) + rb_r[...]) * rg_r[...] + rbe_r[...]
        o = jnp.maximum(ytc + res2, 0.0)
        if last:
            o_ref[0] = jax.lax.dot_general(p_r[...], o, (((0,), (0,)), ((), ())),
                                           preferred_element_type=jnp.float32)
        else:
            o_ref[0] = o

    return body


def _full_spec(shape):
    nd = len(shape)
    return pl.BlockSpec(shape, lambda n: (0,) * nd)


def _run_layer(x, layer, cfg, first, last, g0=None, b0=None):
    ci, co, st, res = cfg
    it = co // 4
    n = x.shape[0]
    t = x.shape[1] // _VP
    to = t // st
    gcn, tcn = layer['gcn'], layer['tcn']
    has_down = 'down_w' in gcn
    res_kind = 'none' if not res else ('conv' if 'res' in layer else 'id')

    pa = jnp.zeros((3, _VP, _VP), jnp.float32).at[:, :_V, :_V].set(gcn['PA'])
    pa = jnp.transpose(pa, (1, 0, 2)).reshape(_VP, 3 * _VP)
    wab = jnp.concatenate([gcn['wa'].reshape(3 * it, ci),
                           gcn['wb'].reshape(3 * it, ci)], axis=0).T
    wdc = jnp.transpose(gcn['wd'].reshape(3, co, ci), (0, 2, 1)).reshape(3 * ci, co)
    ins = [x, pa, wab, wdc,
           gcn['bd'].reshape(3, 1, co),
           gcn['bn_g'].reshape(1, co),
           gcn['bn_b'].reshape(1, co)]
    if has_down:
        ins += [gcn['down_w'].reshape(co, ci).T, gcn['down_b'].reshape(1, co),
                gcn['down_g'].reshape(1, co), gcn['down_bb'].reshape(1, co)]
    wt9 = jnp.transpose(tcn['w'].reshape(co, co, 9), (2, 1, 0))
    if st == 2:
        wt9 = wt9[jnp.asarray([0, 2, 4, 6, 8, 1, 3, 5, 7])]
    wt = wt9.reshape(9 * co, co)
    ins += [wt, tcn['b'].reshape(1, co), tcn['g'].reshape(1, co),
            tcn['be'].reshape(1, co)]
    if res_kind == 'conv':
        r = layer['res']
        ins += [r['w'].reshape(co, ci).T, r['b'].reshape(1, co),
                r['g'].reshape(1, co), r['be'].reshape(1, co)]
    if first:
        ins += [g0, b0]
    if last:
        ins += [jnp.asarray(_P)]

    in_specs = [pl.BlockSpec((1, t * _VP, ci), lambda n: (n, 0, 0))]
    in_specs += [_full_spec(a.shape) for a in ins[1:]]
    if last:
        out_shape = jax.ShapeDtypeStruct((n, 64, co), jnp.float32)
        out_specs = pl.BlockSpec((1, 64, co), lambda n: (n, 0, 0))
    else:
        out_shape = jax.ShapeDtypeStruct((n, to * _VP, co), jnp.float32)
        out_specs = pl.BlockSpec((1, to * _VP, co), lambda n: (n, 0, 0))

    body = _make_body(ci, co, it, t, st, has_down, res_kind, first, last)
    return pl.pallas_call(
        body,
        grid=(n,),
        in_specs=in_specs,
        out_specs=out_specs,
        out_shape=out_shape,
        compiler_params=pltpu.CompilerParams(
            dimension_semantics=("arbitrary",)),
    )(*ins)


def kernel(keypoints, params):
    n = keypoints.shape[0]
    xr = keypoints[..., 0]
    xp = jnp.pad(xr, ((0, 0), (0, 0), (0, 0), (0, _VP - _V)))
    x = jnp.transpose(xp, (0, 2, 3, 1)).reshape(n, 64 * _VP, 2)
    g2 = params['data_bn_g'].reshape(_V, 2)
    b2 = params['data_bn_b'].reshape(_V, 2)
    gv = jnp.pad(g2, ((0, _VP - _V), (0, 0)))
    bv = jnp.pad(b2, ((0, _VP - _V), (0, 0)))
    g0 = jnp.tile(gv, (64, 1))
    b0 = jnp.tile(bv, (64, 1))
    nlayers = len(_CFG)
    for li, (layer, cfg) in enumerate(zip(params['layers'], _CFG)):
        x = _run_layer(x, layer, cfg, first=(li == 0), last=(li == nlayers - 1),
                       g0=g0 if li == 0 else None, b0=b0 if li == 0 else None)
    return x
